# baseline (speedup 1.0000x reference)
import functools

import jax
import jax.numpy as jnp
from jax import lax
from jax.experimental import pallas as pl
from jax.experimental.pallas import tpu as pltpu
from jax.experimental.pallas import tpu_sc as plsc

TOPK = 64
PACK = 4
LANES = 16
UNROLL = 4


def _score_kernel(x_ref, w1_ref, w2_ref, w3_ref, b1_ref, b2_ref, b3_ref,
                  s1_ref, s2_ref, g1_ref, be1_ref, g2_ref, be2_ref, out_ref):
    B, Rb, _ = x_ref.shape
    rows = B * Rb
    x2 = x_ref[...].reshape(rows, 128)

    h1 = jnp.dot(x2.astype(jnp.bfloat16), w1_ref[...].astype(jnp.bfloat16),
                 preferred_element_type=jnp.float32)
    h1 = h1 + b1_ref[...][None, :]
    h1b = h1.reshape(B, Rb, 64)

    def bn_relu(hb, s_ref, g_ref, be_ref, nred):
        sb = jnp.sum(hb, axis=0)
        qb = jnp.sum(hb * hb, axis=0)
        s = s_ref[...]
        m = jnp.dot(sb, s, preferred_element_type=jnp.float32, precision=lax.Precision.HIGHEST) / nred
        q = jnp.dot(qb, s, preferred_element_type=jnp.float32, precision=lax.Precision.HIGHEST) / nred
        inv = 1.0 / jnp.sqrt(q - m * m + 1e-5)
        a = g_ref[...] * inv
        c = be_ref[...] - m * a
        st = s.T
        a_w = jnp.dot(a, st, preferred_element_type=jnp.float32, precision=lax.Precision.HIGHEST)
        c_w = jnp.dot(c, st, preferred_element_type=jnp.float32, precision=lax.Precision.HIGHEST)
        return jnp.maximum(hb * a_w[None, :, :] + c_w[None, :, :], 0.0)

    h1n = bn_relu(h1b, s1_ref, g1_ref, be1_ref, 32.0 * 16.0)

    h2 = jnp.dot(h1n.reshape(rows, 64).astype(jnp.bfloat16),
                 w2_ref[...].astype(jnp.bfloat16),
                 preferred_element_type=jnp.float32)
    h2 = h2 + b2_ref[...][None, :]
    h2n = bn_relu(h2.reshape(B, Rb, 32), s2_ref, g2_ref, be2_ref, 32.0 * 8.0)

    h3 = jnp.dot(h2n.reshape(rows, 32).astype(jnp.bfloat16),
                 w3_ref[...].astype(jnp.bfloat16),
                 preferred_element_type=jnp.float32)
    h3 = h3 + b3_ref[...][None, :]
    sp = jnp.maximum(h3, 0.0) + jnp.log1p(jnp.exp(-jnp.abs(h3)))
    out_ref[...] = jnp.swapaxes(sp.reshape(B, Rb, PACK), 1, 2)


def _scores(x, w1, b1, g1, be1, w2, b2, g2, be2, w3, b3, rb=256,
            interpret=False):
    B, N, C = x.shape
    R = N // PACK
    xr = x.reshape(B, R, C * PACK)

    eye = jnp.eye(PACK, dtype=jnp.float32)
    w1p = jnp.kron(eye, w1.T)
    w2p = jnp.kron(eye, w2.T)
    w3p = jnp.kron(eye, w3.T)
    b1p = jnp.tile(b1, PACK)
    b2p = jnp.tile(b2, PACK)
    b3p = jnp.tile(b3, PACK)
    s1 = jnp.kron(eye, jnp.ones((16, 1), jnp.float32))
    s2 = jnp.kron(eye, jnp.ones((8, 1), jnp.float32))
    g1r = g1.reshape(R, PACK)
    be1r = be1.reshape(R, PACK)
    g2r = g2.reshape(R, PACK)
    be2r = be2.reshape(R, PACK)

    grid = (R // rb,)
    scores = pl.pallas_call(
        _score_kernel,
        grid=grid,
        in_specs=[
            pl.BlockSpec((B, rb, C * PACK), lambda i: (0, i, 0)),
            pl.BlockSpec((C * PACK, 64), lambda i: (0, 0)),
            pl.BlockSpec((64, 32), lambda i: (0, 0)),
            pl.BlockSpec((32, PACK), lambda i: (0, 0)),
            pl.BlockSpec((64,), lambda i: (0,)),
            pl.BlockSpec((32,), lambda i: (0,)),
            pl.BlockSpec((PACK,), lambda i: (0,)),
            pl.BlockSpec((64, PACK), lambda i: (0, 0)),
            pl.BlockSpec((32, PACK), lambda i: (0, 0)),
            pl.BlockSpec((rb, PACK), lambda i: (i, 0)),
            pl.BlockSpec((rb, PACK), lambda i: (i, 0)),
            pl.BlockSpec((rb, PACK), lambda i: (i, 0)),
            pl.BlockSpec((rb, PACK), lambda i: (i, 0)),
        ],
        out_specs=pl.BlockSpec((B, PACK, rb), lambda i: (0, 0, i)),
        out_shape=jax.ShapeDtypeStruct((B, PACK, R), jnp.float32),
        interpret=interpret,
    )(xr, w1p, w2p, w3p, b1p, b2p, b3p, s1, s2, g1r, be1r, g2r, be2r)
    return scores.reshape(B, N)


def _rev(v):
    return lax.rev(v, (0,))


def _ce(a, ai, b, bi):
    sel = a >= b
    return (jnp.where(sel, a, b), jnp.where(sel, ai, bi),
            jnp.where(sel, b, a), jnp.where(sel, bi, ai))


def _merge2(a, ai, b, bi):
    rb, rbi = _rev(b), _rev(bi)
    sel = a >= rb
    lk = jnp.where(sel, a, rb)
    li = jnp.where(sel, ai, rbi)
    hk = jnp.where(sel, rb, a)
    hi = jnp.where(sel, rbi, ai)
    lk, li = plsc.sort_key_val(lk, li, descending=True)
    hk, hi = plsc.sort_key_val(hk, hi, descending=True)
    return [lk, hk], [li, hi]


def _merge4(A, Ai, B, Bi):
    x = [A[0], A[1], _rev(B[1]), _rev(B[0])]
    xi = [Ai[0], Ai[1], _rev(Bi[1]), _rev(Bi[0])]
    t0, t0i, b0, b0i = _ce(x[0], xi[0], x[2], xi[2])
    t1, t1i, b1, b1i = _ce(x[1], xi[1], x[3], xi[3])
    u0, u0i, u1, u1i = _ce(t0, t0i, t1, t1i)
    v0, v0i, v1, v1i = _ce(b0, b0i, b1, b1i)
    ok, oi = [], []
    for kk, ii in ((u0, u0i), (u1, u1i), (v0, v0i), (v1, v1i)):
        kk, ii = plsc.sort_key_val(kk, ii, descending=True)
        ok.append(kk)
        oi.append(ii)
    return ok, oi


def _top64of128(bufk, bufi):
    sk, si = [], []
    for j in range(8):
        kk, ii = plsc.sort_key_val(bufk[j], bufi[j], descending=True)
        sk.append(kk)
        si.append(ii)
    m32k, m32i = [], []
    for j in range(4):
        K, I = _merge2(sk[2 * j], si[2 * j], sk[2 * j + 1], si[2 * j + 1])
        m32k.append(K)
        m32i.append(I)
    A, Ai = _merge4(m32k[0], m32i[0], m32k[1], m32i[1])
    B, Bi = _merge4(m32k[2], m32i[2], m32k[3], m32i[3])
    rB = [_rev(B[3]), _rev(B[2]), _rev(B[1]), _rev(B[0])]
    rBi = [_rev(Bi[3]), _rev(Bi[2]), _rev(Bi[1]), _rev(Bi[0])]
    ck, ci = [], []
    for j in range(4):
        sel = A[j] >= rB[j]
        ck.append(jnp.where(sel, A[j], rB[j]))
        ci.append(jnp.where(sel, Ai[j], rBi[j]))
    return ck, ci


def _sort64(ck, ci):
    t0, t0i, b0, b0i = _ce(ck[0], ci[0], ck[2], ci[2])
    t1, t1i, b1, b1i = _ce(ck[1], ci[1], ck[3], ci[3])
    u0, u0i, u1, u1i = _ce(t0, t0i, t1, t1i)
    v0, v0i, v1, v1i = _ce(b0, b0i, b1, b1i)
    ok, oi = [], []
    for kk, ii in ((u0, u0i), (u1, u1i), (v0, v0i), (v1, v1i)):
        kk, ii = plsc.sort_key_val(kk, ii, descending=True)
        ok.append(kk)
        oi.append(ii)
    return ok, oi


def _topk_sc(flat):
    B, N = flat.shape
    nv_total = N // LANES
    mesh = plsc.VectorSubcoreMesh(core_axis_name="c", subcore_axis_name="s")

    @functools.partial(
        pl.kernel,
        mesh=mesh,
        out_type=[jax.ShapeDtypeStruct((B, TOPK), jnp.float32),
                  jax.ShapeDtypeStruct((B, TOPK), jnp.int32)],
        scratch_types=[pltpu.VMEM((N,), jnp.float32),
                       pltpu.VMEM((8 * LANES,), jnp.float32),
                       pltpu.VMEM((8 * LANES,), jnp.int32),
                       pltpu.VMEM((TOPK,), jnp.float32),
                       pltpu.VMEM((TOPK,), jnp.int32)],
        compiler_params=pltpu.CompilerParams(needs_layout_passes=False),
    )
    def tk(scores_hbm, vals_hbm, idx_hbm, row_v, bufk_v, bufi_v,
           outv_v, outi_v):
        row = lax.axis_index("s") * 2 + lax.axis_index("c")
        pltpu.sync_copy(scores_hbm.at[row], row_v)

        neg = jnp.full((LANES,), -1.0, jnp.float32)
        zero_i = jnp.zeros((LANES,), jnp.int32)
        for j in range(8):
            bufk_v[pl.ds(LANES * j, LANES)] = neg
            bufi_v[pl.ds(LANES * j, LANES)] = zero_i
        iota = lax.iota(jnp.int32, LANES)

        def load_buf():
            bk = [bufk_v[pl.ds(LANES * j, LANES)] for j in range(8)]
            bi = [bufi_v[pl.ds(LANES * j, LANES)] for j in range(8)]
            return bk, bi

        def rebuild(nv_t):
            bk, bi = load_buf()
            ck, ci = _top64of128(bk, bi)
            for j in range(4):
                bufk_v[pl.ds(LANES * j, LANES)] = ck[j]
                bufi_v[pl.ds(LANES * j, LANES)] = ci[j]
            for j in range(4, 8):
                bufk_v[pl.ds(LANES * j, LANES)] = neg
                bufi_v[pl.ds(LANES * j, LANES)] = zero_i
            m = jnp.minimum(jnp.minimum(ck[0], ck[1]),
                            jnp.minimum(ck[2], ck[3]))
            s, _ = plsc.sort_key_val(m, m, descending=False)
            t2 = s[0]
            return jnp.int32(4), t2

        def body(i, carry):
            base = i * (LANES * UNROLL)
            vs = [row_v[pl.ds(base + LANES * u, LANES)]
                  for u in range(UNROLL)]
            t_old = carry[1]
            ms = [v > t_old for v in vs]
            anym = ms[0]
            for u in range(1, UNROLL):
                anym = anym | ms[u]

            def any_lanes(m):
                return plsc.all_reduce_population_count(m)[0] > 0

            def slow(c):
                nv, t = c
                for u in range(UNROLL):
                    def do_append(nv_t, u=u):
                        nv2, t2 = nv_t
                        bufk_v[pl.ds(nv2 * LANES, LANES)] = jnp.where(
                            ms[u], vs[u], -1.0)
                        bufi_v[pl.ds(nv2 * LANES, LANES)] = jnp.where(
                            ms[u], iota + (base + LANES * u), 0)
                        nv3 = nv2 + 1
                        return lax.cond(nv3 == 8, rebuild,
                                        lambda a: a, (nv3, t2))
                    nv, t = lax.cond(any_lanes(ms[u]), do_append,
                                     lambda a: a, (nv, t))
                return nv, t

            return lax.cond(any_lanes(anym), slow, lambda c: c, carry)

        nv, t = lax.fori_loop(0, nv_total // UNROLL, body,
                              (jnp.int32(4), jnp.float32(-1.0)))

        bk, bi = load_buf()
        ck, ci = _top64of128(bk, bi)
        sk, si = _sort64(ck, ci)
        rmask = jnp.int32(N // PACK - 1)
        rshift = (N // PACK).bit_length() - 1
        for j in range(4):
            outv_v[pl.ds(LANES * j, LANES)] = sk[j]
            outi_v[pl.ds(LANES * j, LANES)] = (
                (si[j] & rmask) * PACK
                + jnp.right_shift(si[j], jnp.int32(rshift)))
        pltpu.sync_copy(outv_v, vals_hbm.at[row])
        pltpu.sync_copy(outi_v, idx_hbm.at[row])

    return tk(flat)


def kernel(x, w1, b1, g1, be1, w2, b2, g2, be2, w3, b3):
    flat = _scores(x, w1, b1, g1, be1, w2, b2, g2, be2, w3, b3)
    _, idx0 = _topk_sc(flat)

    B = x.shape[0]
    idx_s = jnp.sort(idx0, axis=1)
    cand = idx_s.reshape(-1)
    xg = x[:, cand, :]
    g1g, be1g = g1[cand], be1[cand]
    g2g, be2g = g2[cand], be2[cand]

    def bn(h, g, b, eps=1e-5):
        m = jnp.mean(h, axis=(0, 2), keepdims=True)
        v = jnp.var(h, axis=(0, 2), keepdims=True)
        return (h - m) / jnp.sqrt(v + eps) * g[None, :, None] + b[None, :, None]

    h = xg @ w1.T + b1
    h = jax.nn.relu(bn(h, g1g, be1g))
    h = h @ w2.T + b2
    h = jax.nn.relu(bn(h, g2g, be2g))
    h = h @ w3.T + b3
    h = jax.nn.softplus(h)
    s = h.reshape(B, B, TOPK)
    rows = jnp.arange(B)
    block = s[rows, rows]
    vals, order = jax.lax.top_k(block, TOPK)
    idx = jnp.take_along_axis(idx_s, order, axis=1)
    return vals, idx


# speedup vs baseline: 1.0375x; 1.0375x over previous
import functools

import jax
import jax.numpy as jnp
from jax import lax
from jax.experimental import pallas as pl
from jax.experimental.pallas import tpu as pltpu
from jax.experimental.pallas import tpu_sc as plsc

TOPK = 64
PACK = 4
LANES = 16
UNROLL = 4


def _score_kernel(x_ref, w1_ref, w2_ref, w3_ref, b1_ref, b2_ref, b3_ref,
                  s1_ref, s2_ref, g1_ref, be1_ref, g2_ref, be2_ref, out_ref):
    B, Rb, _ = x_ref.shape
    rows = B * Rb
    x2 = x_ref[...].reshape(rows, 128)

    h1 = jnp.dot(x2.astype(jnp.bfloat16), w1_ref[...].astype(jnp.bfloat16),
                 preferred_element_type=jnp.float32)
    h1 = h1 + b1_ref[...][None, :]
    h1b = h1.reshape(B, Rb, 64)

    def bn_relu(hb, s_ref, g_ref, be_ref, nred):
        sb = jnp.sum(hb, axis=0)
        qb = jnp.sum(hb * hb, axis=0)
        s = s_ref[...]
        m = jnp.dot(sb, s, preferred_element_type=jnp.float32, precision=lax.Precision.HIGHEST) / nred
        q = jnp.dot(qb, s, preferred_element_type=jnp.float32, precision=lax.Precision.HIGHEST) / nred
        inv = 1.0 / jnp.sqrt(q - m * m + 1e-5)
        a = g_ref[...] * inv
        c = be_ref[...] - m * a
        st = s.T
        a_w = jnp.dot(a, st, preferred_element_type=jnp.float32, precision=lax.Precision.HIGHEST)
        c_w = jnp.dot(c, st, preferred_element_type=jnp.float32, precision=lax.Precision.HIGHEST)
        return jnp.maximum(hb * a_w[None, :, :] + c_w[None, :, :], 0.0)

    h1n = bn_relu(h1b, s1_ref, g1_ref, be1_ref, 32.0 * 16.0)

    h2 = jnp.dot(h1n.reshape(rows, 64).astype(jnp.bfloat16),
                 w2_ref[...].astype(jnp.bfloat16),
                 preferred_element_type=jnp.float32)
    h2 = h2 + b2_ref[...][None, :]
    h2n = bn_relu(h2.reshape(B, Rb, 32), s2_ref, g2_ref, be2_ref, 32.0 * 8.0)

    h3 = jnp.dot(h2n.reshape(rows, 32).astype(jnp.bfloat16),
                 w3_ref[...].astype(jnp.bfloat16),
                 preferred_element_type=jnp.float32)
    h3 = h3 + b3_ref[...][None, :]
    out_ref[...] = jnp.swapaxes(h3.reshape(B, Rb, PACK), 1, 2)


def _scores(x, w1, b1, g1, be1, w2, b2, g2, be2, w3, b3, rb=256,
            interpret=False):
    B, N, C = x.shape
    R = N // PACK
    xr = x.reshape(B, R, C * PACK)

    eye = jnp.eye(PACK, dtype=jnp.float32)
    w1p = jnp.kron(eye, w1.T)
    w2p = jnp.kron(eye, w2.T)
    w3p = jnp.kron(eye, w3.T)
    b1p = jnp.tile(b1, PACK)
    b2p = jnp.tile(b2, PACK)
    b3p = jnp.tile(b3, PACK)
    s1 = jnp.kron(eye, jnp.ones((16, 1), jnp.float32))
    s2 = jnp.kron(eye, jnp.ones((8, 1), jnp.float32))
    g1r = g1.reshape(R, PACK)
    be1r = be1.reshape(R, PACK)
    g2r = g2.reshape(R, PACK)
    be2r = be2.reshape(R, PACK)

    grid = (R // rb,)
    scores = pl.pallas_call(
        _score_kernel,
        grid=grid,
        in_specs=[
            pl.BlockSpec((B, rb, C * PACK), lambda i: (0, i, 0)),
            pl.BlockSpec((C * PACK, 64), lambda i: (0, 0)),
            pl.BlockSpec((64, 32), lambda i: (0, 0)),
            pl.BlockSpec((32, PACK), lambda i: (0, 0)),
            pl.BlockSpec((64,), lambda i: (0,)),
            pl.BlockSpec((32,), lambda i: (0,)),
            pl.BlockSpec((PACK,), lambda i: (0,)),
            pl.BlockSpec((64, PACK), lambda i: (0, 0)),
            pl.BlockSpec((32, PACK), lambda i: (0, 0)),
            pl.BlockSpec((rb, PACK), lambda i: (i, 0)),
            pl.BlockSpec((rb, PACK), lambda i: (i, 0)),
            pl.BlockSpec((rb, PACK), lambda i: (i, 0)),
            pl.BlockSpec((rb, PACK), lambda i: (i, 0)),
        ],
        out_specs=pl.BlockSpec((B, PACK, rb), lambda i: (0, 0, i)),
        out_shape=jax.ShapeDtypeStruct((B, PACK, R), jnp.float32),
        interpret=interpret,
    )(xr, w1p, w2p, w3p, b1p, b2p, b3p, s1, s2, g1r, be1r, g2r, be2r)
    return scores.reshape(B, N)


def _rev(v):
    return lax.rev(v, (0,))


def _ce(a, ai, b, bi):
    sel = a >= b
    return (jnp.where(sel, a, b), jnp.where(sel, ai, bi),
            jnp.where(sel, b, a), jnp.where(sel, bi, ai))


def _merge2(a, ai, b, bi):
    rb, rbi = _rev(b), _rev(bi)
    sel = a >= rb
    lk = jnp.where(sel, a, rb)
    li = jnp.where(sel, ai, rbi)
    hk = jnp.where(sel, rb, a)
    hi = jnp.where(sel, rbi, ai)
    lk, li = plsc.sort_key_val(lk, li, descending=True)
    hk, hi = plsc.sort_key_val(hk, hi, descending=True)
    return [lk, hk], [li, hi]


def _merge4(A, Ai, B, Bi):
    x = [A[0], A[1], _rev(B[1]), _rev(B[0])]
    xi = [Ai[0], Ai[1], _rev(Bi[1]), _rev(Bi[0])]
    t0, t0i, b0, b0i = _ce(x[0], xi[0], x[2], xi[2])
    t1, t1i, b1, b1i = _ce(x[1], xi[1], x[3], xi[3])
    u0, u0i, u1, u1i = _ce(t0, t0i, t1, t1i)
    v0, v0i, v1, v1i = _ce(b0, b0i, b1, b1i)
    ok, oi = [], []
    for kk, ii in ((u0, u0i), (u1, u1i), (v0, v0i), (v1, v1i)):
        kk, ii = plsc.sort_key_val(kk, ii, descending=True)
        ok.append(kk)
        oi.append(ii)
    return ok, oi


def _top64of128(bufk, bufi):
    sk, si = [], []
    for j in range(8):
        kk, ii = plsc.sort_key_val(bufk[j], bufi[j], descending=True)
        sk.append(kk)
        si.append(ii)
    m32k, m32i = [], []
    for j in range(4):
        K, I = _merge2(sk[2 * j], si[2 * j], sk[2 * j + 1], si[2 * j + 1])
        m32k.append(K)
        m32i.append(I)
    A, Ai = _merge4(m32k[0], m32i[0], m32k[1], m32i[1])
    B, Bi = _merge4(m32k[2], m32i[2], m32k[3], m32i[3])
    rB = [_rev(B[3]), _rev(B[2]), _rev(B[1]), _rev(B[0])]
    rBi = [_rev(Bi[3]), _rev(Bi[2]), _rev(Bi[1]), _rev(Bi[0])]
    ck, ci = [], []
    for j in range(4):
        sel = A[j] >= rB[j]
        ck.append(jnp.where(sel, A[j], rB[j]))
        ci.append(jnp.where(sel, Ai[j], rBi[j]))
    return ck, ci


def _sort64(ck, ci):
    t0, t0i, b0, b0i = _ce(ck[0], ci[0], ck[2], ci[2])
    t1, t1i, b1, b1i = _ce(ck[1], ci[1], ck[3], ci[3])
    u0, u0i, u1, u1i = _ce(t0, t0i, t1, t1i)
    v0, v0i, v1, v1i = _ce(b0, b0i, b1, b1i)
    ok, oi = [], []
    for kk, ii in ((u0, u0i), (u1, u1i), (v0, v0i), (v1, v1i)):
        kk, ii = plsc.sort_key_val(kk, ii, descending=True)
        ok.append(kk)
        oi.append(ii)
    return ok, oi


def _topk_sc(flat):
    B, N = flat.shape
    nv_total = N // LANES
    mesh = plsc.VectorSubcoreMesh(core_axis_name="c", subcore_axis_name="s")

    @functools.partial(
        pl.kernel,
        mesh=mesh,
        out_type=[jax.ShapeDtypeStruct((B, TOPK), jnp.float32),
                  jax.ShapeDtypeStruct((B, TOPK), jnp.int32)],
        scratch_types=[pltpu.VMEM((N,), jnp.float32),
                       pltpu.VMEM((8 * LANES,), jnp.float32),
                       pltpu.VMEM((8 * LANES,), jnp.int32),
                       pltpu.VMEM((TOPK,), jnp.float32),
                       pltpu.VMEM((TOPK,), jnp.int32)],
        compiler_params=pltpu.CompilerParams(needs_layout_passes=False),
    )
    def tk(scores_hbm, vals_hbm, idx_hbm, row_v, bufk_v, bufi_v,
           outv_v, outi_v):
        row = lax.axis_index("s") * 2 + lax.axis_index("c")
        pltpu.sync_copy(scores_hbm.at[row], row_v)

        neg = jnp.full((LANES,), -3.0e38, jnp.float32)
        zero_i = jnp.zeros((LANES,), jnp.int32)
        for j in range(8):
            bufk_v[pl.ds(LANES * j, LANES)] = neg
            bufi_v[pl.ds(LANES * j, LANES)] = zero_i
        iota = lax.iota(jnp.int32, LANES)

        def load_buf():
            bk = [bufk_v[pl.ds(LANES * j, LANES)] for j in range(8)]
            bi = [bufi_v[pl.ds(LANES * j, LANES)] for j in range(8)]
            return bk, bi

        def rebuild(nv_t):
            bk, bi = load_buf()
            ck, ci = _top64of128(bk, bi)
            for j in range(4):
                bufk_v[pl.ds(LANES * j, LANES)] = ck[j]
                bufi_v[pl.ds(LANES * j, LANES)] = ci[j]
            for j in range(4, 8):
                bufk_v[pl.ds(LANES * j, LANES)] = neg
                bufi_v[pl.ds(LANES * j, LANES)] = zero_i
            m = jnp.minimum(jnp.minimum(ck[0], ck[1]),
                            jnp.minimum(ck[2], ck[3]))
            s, _ = plsc.sort_key_val(m, m, descending=False)
            t2 = s[0]
            return jnp.int32(4), t2

        def body(i, carry):
            base = i * (LANES * UNROLL)
            vs = [row_v[pl.ds(base + LANES * u, LANES)]
                  for u in range(UNROLL)]
            t_old = carry[1]
            ms = [v > t_old for v in vs]
            anym = ms[0]
            for u in range(1, UNROLL):
                anym = anym | ms[u]

            def any_lanes(m):
                return plsc.all_reduce_population_count(m)[0] > 0

            def slow(c):
                nv, t = c
                for u in range(UNROLL):
                    def do_append(nv_t, u=u):
                        nv2, t2 = nv_t
                        bufk_v[pl.ds(nv2 * LANES, LANES)] = jnp.where(
                            ms[u], vs[u], -3.0e38)
                        bufi_v[pl.ds(nv2 * LANES, LANES)] = jnp.where(
                            ms[u], iota + (base + LANES * u), 0)
                        nv3 = nv2 + 1
                        return lax.cond(nv3 == 8, rebuild,
                                        lambda a: a, (nv3, t2))
                    nv, t = lax.cond(any_lanes(ms[u]), do_append,
                                     lambda a: a, (nv, t))
                return nv, t

            return lax.cond(any_lanes(anym), slow, lambda c: c, carry)

        nv, t = lax.fori_loop(0, nv_total // UNROLL, body,
                              (jnp.int32(4), jnp.float32(-3.0e38)))

        bk, bi = load_buf()
        ck, ci = _top64of128(bk, bi)
        sk, si = _sort64(ck, ci)
        rmask = jnp.int32(N // PACK - 1)
        rshift = (N // PACK).bit_length() - 1
        for j in range(4):
            outv_v[pl.ds(LANES * j, LANES)] = sk[j]
            outi_v[pl.ds(LANES * j, LANES)] = (
                (si[j] & rmask) * PACK
                + jnp.right_shift(si[j], jnp.int32(rshift)))
        pltpu.sync_copy(outv_v, vals_hbm.at[row])
        pltpu.sync_copy(outi_v, idx_hbm.at[row])

    return tk(flat)


def kernel(x, w1, b1, g1, be1, w2, b2, g2, be2, w3, b3):
    flat = _scores(x, w1, b1, g1, be1, w2, b2, g2, be2, w3, b3)
    _, idx0 = _topk_sc(flat)

    B = x.shape[0]
    idx_s = jnp.sort(idx0, axis=1)
    cand = idx_s.reshape(-1)
    xg = x[:, cand, :]
    g1g, be1g = g1[cand], be1[cand]
    g2g, be2g = g2[cand], be2[cand]

    def bn(h, g, b, eps=1e-5):
        m = jnp.mean(h, axis=(0, 2), keepdims=True)
        v = jnp.var(h, axis=(0, 2), keepdims=True)
        return (h - m) / jnp.sqrt(v + eps) * g[None, :, None] + b[None, :, None]

    h = xg @ w1.T + b1
    h = jax.nn.relu(bn(h, g1g, be1g))
    h = h @ w2.T + b2
    h = jax.nn.relu(bn(h, g2g, be2g))
    h = h @ w3.T + b3
    h = jax.nn.softplus(h)
    s = h.reshape(B, B, TOPK)
    rows = jnp.arange(B)
    block = s[rows, rows]
    vals, order = jax.lax.top_k(block, TOPK)
    idx = jnp.take_along_axis(idx_s, order, axis=1)
    return vals, idx


# speedup vs baseline: 1.0438x; 1.0061x over previous
import functools

import jax
import jax.numpy as jnp
from jax import lax
from jax.experimental import pallas as pl
from jax.experimental.pallas import tpu as pltpu
from jax.experimental.pallas import tpu_sc as plsc

TOPK = 64
PACK = 4
LANES = 16
UNROLL = 4


def _score_kernel(x_ref, w1_ref, w2_ref, w3_ref, b1_ref, b2_ref, b3_ref,
                  s1_ref, s2_ref, g1_ref, be1_ref, g2_ref, be2_ref, out_ref):
    B, Rb, _ = x_ref.shape
    rows = B * Rb
    x2 = x_ref[...].reshape(rows, 128)

    h1 = jnp.dot(x2.astype(jnp.bfloat16), w1_ref[...].astype(jnp.bfloat16),
                 preferred_element_type=jnp.float32)
    h1 = h1 + b1_ref[...][None, :]
    h1b = h1.reshape(B, Rb, 64)

    def bn_relu(hb, s_ref, g_ref, be_ref, nred):
        sb = jnp.sum(hb, axis=0)
        qb = jnp.sum(hb * hb, axis=0)
        s = s_ref[...]
        m = jnp.dot(sb, s, preferred_element_type=jnp.float32, precision=lax.Precision.HIGHEST) / nred
        q = jnp.dot(qb, s, preferred_element_type=jnp.float32, precision=lax.Precision.HIGHEST) / nred
        inv = 1.0 / jnp.sqrt(q - m * m + 1e-5)
        a = g_ref[...] * inv
        c = be_ref[...] - m * a
        st = s.T
        a_w = jnp.dot(a, st, preferred_element_type=jnp.float32, precision=lax.Precision.HIGHEST)
        c_w = jnp.dot(c, st, preferred_element_type=jnp.float32, precision=lax.Precision.HIGHEST)
        return jnp.maximum(hb * a_w[None, :, :] + c_w[None, :, :], 0.0)

    h1n = bn_relu(h1b, s1_ref, g1_ref, be1_ref, 32.0 * 16.0)

    h2 = jnp.dot(h1n.reshape(rows, 64).astype(jnp.bfloat16),
                 w2_ref[...].astype(jnp.bfloat16),
                 preferred_element_type=jnp.float32)
    h2 = h2 + b2_ref[...][None, :]
    h2n = bn_relu(h2.reshape(B, Rb, 32), s2_ref, g2_ref, be2_ref, 32.0 * 8.0)

    h3 = jnp.dot(h2n.reshape(rows, 32).astype(jnp.bfloat16),
                 w3_ref[...].astype(jnp.bfloat16),
                 preferred_element_type=jnp.float32)
    h3 = h3 + b3_ref[...][None, :]
    out_ref[...] = jnp.swapaxes(h3.reshape(B, Rb, PACK), 1, 2)


def _scores(x, w1, b1, g1, be1, w2, b2, g2, be2, w3, b3, rb=512,
            interpret=False):
    B, N, C = x.shape
    R = N // PACK
    xr = x.reshape(B, R, C * PACK)

    eye = jnp.eye(PACK, dtype=jnp.float32)
    w1p = jnp.kron(eye, w1.T)
    w2p = jnp.kron(eye, w2.T)
    w3p = jnp.kron(eye, w3.T)
    b1p = jnp.tile(b1, PACK)
    b2p = jnp.tile(b2, PACK)
    b3p = jnp.tile(b3, PACK)
    s1 = jnp.kron(eye, jnp.ones((16, 1), jnp.float32))
    s2 = jnp.kron(eye, jnp.ones((8, 1), jnp.float32))
    g1r = g1.reshape(R, PACK)
    be1r = be1.reshape(R, PACK)
    g2r = g2.reshape(R, PACK)
    be2r = be2.reshape(R, PACK)

    grid = (R // rb,)
    scores = pl.pallas_call(
        _score_kernel,
        grid=grid,
        in_specs=[
            pl.BlockSpec((B, rb, C * PACK), lambda i: (0, i, 0)),
            pl.BlockSpec((C * PACK, 64), lambda i: (0, 0)),
            pl.BlockSpec((64, 32), lambda i: (0, 0)),
            pl.BlockSpec((32, PACK), lambda i: (0, 0)),
            pl.BlockSpec((64,), lambda i: (0,)),
            pl.BlockSpec((32,), lambda i: (0,)),
            pl.BlockSpec((PACK,), lambda i: (0,)),
            pl.BlockSpec((64, PACK), lambda i: (0, 0)),
            pl.BlockSpec((32, PACK), lambda i: (0, 0)),
            pl.BlockSpec((rb, PACK), lambda i: (i, 0)),
            pl.BlockSpec((rb, PACK), lambda i: (i, 0)),
            pl.BlockSpec((rb, PACK), lambda i: (i, 0)),
            pl.BlockSpec((rb, PACK), lambda i: (i, 0)),
        ],
        out_specs=pl.BlockSpec((B, PACK, rb), lambda i: (0, 0, i)),
        out_shape=jax.ShapeDtypeStruct((B, PACK, R), jnp.float32),
        interpret=interpret,
    )(xr, w1p, w2p, w3p, b1p, b2p, b3p, s1, s2, g1r, be1r, g2r, be2r)
    return scores.reshape(B, N)


def _rev(v):
    return lax.rev(v, (0,))


def _ce(a, ai, b, bi):
    sel = a >= b
    return (jnp.where(sel, a, b), jnp.where(sel, ai, bi),
            jnp.where(sel, b, a), jnp.where(sel, bi, ai))


def _merge2(a, ai, b, bi):
    rb, rbi = _rev(b), _rev(bi)
    sel = a >= rb
    lk = jnp.where(sel, a, rb)
    li = jnp.where(sel, ai, rbi)
    hk = jnp.where(sel, rb, a)
    hi = jnp.where(sel, rbi, ai)
    lk, li = plsc.sort_key_val(lk, li, descending=True)
    hk, hi = plsc.sort_key_val(hk, hi, descending=True)
    return [lk, hk], [li, hi]


def _merge4(A, Ai, B, Bi):
    x = [A[0], A[1], _rev(B[1]), _rev(B[0])]
    xi = [Ai[0], Ai[1], _rev(Bi[1]), _rev(Bi[0])]
    t0, t0i, b0, b0i = _ce(x[0], xi[0], x[2], xi[2])
    t1, t1i, b1, b1i = _ce(x[1], xi[1], x[3], xi[3])
    u0, u0i, u1, u1i = _ce(t0, t0i, t1, t1i)
    v0, v0i, v1, v1i = _ce(b0, b0i, b1, b1i)
    ok, oi = [], []
    for kk, ii in ((u0, u0i), (u1, u1i), (v0, v0i), (v1, v1i)):
        kk, ii = plsc.sort_key_val(kk, ii, descending=True)
        ok.append(kk)
        oi.append(ii)
    return ok, oi


def _top64of128(bufk, bufi):
    sk, si = [], []
    for j in range(8):
        kk, ii = plsc.sort_key_val(bufk[j], bufi[j], descending=True)
        sk.append(kk)
        si.append(ii)
    m32k, m32i = [], []
    for j in range(4):
        K, I = _merge2(sk[2 * j], si[2 * j], sk[2 * j + 1], si[2 * j + 1])
        m32k.append(K)
        m32i.append(I)
    A, Ai = _merge4(m32k[0], m32i[0], m32k[1], m32i[1])
    B, Bi = _merge4(m32k[2], m32i[2], m32k[3], m32i[3])
    rB = [_rev(B[3]), _rev(B[2]), _rev(B[1]), _rev(B[0])]
    rBi = [_rev(Bi[3]), _rev(Bi[2]), _rev(Bi[1]), _rev(Bi[0])]
    ck, ci = [], []
    for j in range(4):
        sel = A[j] >= rB[j]
        ck.append(jnp.where(sel, A[j], rB[j]))
        ci.append(jnp.where(sel, Ai[j], rBi[j]))
    return ck, ci


def _sort64(ck, ci):
    t0, t0i, b0, b0i = _ce(ck[0], ci[0], ck[2], ci[2])
    t1, t1i, b1, b1i = _ce(ck[1], ci[1], ck[3], ci[3])
    u0, u0i, u1, u1i = _ce(t0, t0i, t1, t1i)
    v0, v0i, v1, v1i = _ce(b0, b0i, b1, b1i)
    ok, oi = [], []
    for kk, ii in ((u0, u0i), (u1, u1i), (v0, v0i), (v1, v1i)):
        kk, ii = plsc.sort_key_val(kk, ii, descending=True)
        ok.append(kk)
        oi.append(ii)
    return ok, oi


def _topk_sc(flat):
    B, N = flat.shape
    nv_total = N // LANES
    mesh = plsc.VectorSubcoreMesh(core_axis_name="c", subcore_axis_name="s")

    @functools.partial(
        pl.kernel,
        mesh=mesh,
        out_type=[jax.ShapeDtypeStruct((B, TOPK), jnp.float32),
                  jax.ShapeDtypeStruct((B, TOPK), jnp.int32)],
        scratch_types=[pltpu.VMEM((N,), jnp.float32),
                       pltpu.VMEM((8 * LANES,), jnp.float32),
                       pltpu.VMEM((8 * LANES,), jnp.int32),
                       pltpu.VMEM((TOPK,), jnp.float32),
                       pltpu.VMEM((TOPK,), jnp.int32)],
        compiler_params=pltpu.CompilerParams(needs_layout_passes=False),
    )
    def tk(scores_hbm, vals_hbm, idx_hbm, row_v, bufk_v, bufi_v,
           outv_v, outi_v):
        row = lax.axis_index("s") * 2 + lax.axis_index("c")
        pltpu.sync_copy(scores_hbm.at[row], row_v)

        neg = jnp.full((LANES,), -3.0e38, jnp.float32)
        zero_i = jnp.zeros((LANES,), jnp.int32)
        for j in range(8):
            bufk_v[pl.ds(LANES * j, LANES)] = neg
            bufi_v[pl.ds(LANES * j, LANES)] = zero_i
        iota = lax.iota(jnp.int32, LANES)

        def load_buf():
            bk = [bufk_v[pl.ds(LANES * j, LANES)] for j in range(8)]
            bi = [bufi_v[pl.ds(LANES * j, LANES)] for j in range(8)]
            return bk, bi

        def rebuild(nv_t):
            bk, bi = load_buf()
            ck, ci = _top64of128(bk, bi)
            for j in range(4):
                bufk_v[pl.ds(LANES * j, LANES)] = ck[j]
                bufi_v[pl.ds(LANES * j, LANES)] = ci[j]
            for j in range(4, 8):
                bufk_v[pl.ds(LANES * j, LANES)] = neg
                bufi_v[pl.ds(LANES * j, LANES)] = zero_i
            m = jnp.minimum(jnp.minimum(ck[0], ck[1]),
                            jnp.minimum(ck[2], ck[3]))
            s, _ = plsc.sort_key_val(m, m, descending=False)
            t2 = s[0]
            return jnp.int32(4), t2

        def body(i, carry):
            base = i * (LANES * UNROLL)
            vs = [row_v[pl.ds(base + LANES * u, LANES)]
                  for u in range(UNROLL)]
            t_old = carry[1]
            ms = [v > t_old for v in vs]
            anym = ms[0]
            for u in range(1, UNROLL):
                anym = anym | ms[u]

            def any_lanes(m):
                return plsc.all_reduce_population_count(m)[0] > 0

            def slow(c):
                nv, t = c
                for u in range(UNROLL):
                    def do_append(nv_t, u=u):
                        nv2, t2 = nv_t
                        bufk_v[pl.ds(nv2 * LANES, LANES)] = jnp.where(
                            ms[u], vs[u], -3.0e38)
                        bufi_v[pl.ds(nv2 * LANES, LANES)] = jnp.where(
                            ms[u], iota + (base + LANES * u), 0)
                        nv3 = nv2 + 1
                        return lax.cond(nv3 == 8, rebuild,
                                        lambda a: a, (nv3, t2))
                    nv, t = lax.cond(any_lanes(ms[u]), do_append,
                                     lambda a: a, (nv, t))
                return nv, t

            return lax.cond(any_lanes(anym), slow, lambda c: c, carry)

        nv, t = lax.fori_loop(0, nv_total // UNROLL, body,
                              (jnp.int32(4), jnp.float32(-3.0e38)))

        bk, bi = load_buf()
        ck, ci = _top64of128(bk, bi)
        sk, si = _sort64(ck, ci)
        rmask = jnp.int32(N // PACK - 1)
        rshift = (N // PACK).bit_length() - 1
        for j in range(4):
            outv_v[pl.ds(LANES * j, LANES)] = sk[j]
            outi_v[pl.ds(LANES * j, LANES)] = (
                (si[j] & rmask) * PACK
                + jnp.right_shift(si[j], jnp.int32(rshift)))
        pltpu.sync_copy(outv_v, vals_hbm.at[row])
        pltpu.sync_copy(outi_v, idx_hbm.at[row])

    return tk(flat)


def kernel(x, w1, b1, g1, be1, w2, b2, g2, be2, w3, b3):
    flat = _scores(x, w1, b1, g1, be1, w2, b2, g2, be2, w3, b3)
    _, idx0 = _topk_sc(flat)

    B = x.shape[0]
    idx_s = jnp.sort(idx0, axis=1)
    cand = idx_s.reshape(-1)
    xg = x[:, cand, :]
    g1g, be1g = g1[cand], be1[cand]
    g2g, be2g = g2[cand], be2[cand]

    def bn(h, g, b, eps=1e-5):
        m = jnp.mean(h, axis=(0, 2), keepdims=True)
        v = jnp.var(h, axis=(0, 2), keepdims=True)
        return (h - m) / jnp.sqrt(v + eps) * g[None, :, None] + b[None, :, None]

    h = xg @ w1.T + b1
    h = jax.nn.relu(bn(h, g1g, be1g))
    h = h @ w2.T + b2
    h = jax.nn.relu(bn(h, g2g, be2g))
    h = h @ w3.T + b3
    h = jax.nn.softplus(h)
    s = h.reshape(B, B, TOPK)
    rows = jnp.arange(B)
    block = s[rows, rows]
    vals, order = jax.lax.top_k(block, TOPK)
    idx = jnp.take_along_axis(idx_s, order, axis=1)
    return vals, idx
